# baseline (speedup 1.0000x reference)
import functools

import jax
import jax.numpy as jnp
from jax import lax
from jax.experimental import pallas as pl
from jax.experimental.pallas import tpu as pltpu
from jax.experimental.pallas import tpu_sc as plsc

_NEG_INF = float("-inf")
_BIG = 1 << 30
_SUB = 8


def _dlr_body(xt_ref, y_ref, o_ref):
    xt = xt_ref[...]
    yv = y_ref[...][None, :]
    C, B = xt.shape
    n_chunks = C // _SUB

    subl = jax.lax.broadcasted_iota(jnp.int32, (_SUB, B), 0)
    neg = jnp.full((_SUB, B), _NEG_INF, dtype=jnp.float32)

    ch = xt[:_SUB, :]
    m0, m1, m2 = ch, neg, neg
    cacc = jnp.where(subl == yv, ch, _NEG_INF)

    for k in range(1, n_chunks):
        ch = xt[k * _SUB:(k + 1) * _SUB, :]
        cacc = jnp.where(subl == yv - (k * _SUB), ch, cacc)
        t1 = jnp.minimum(m0, ch)
        m0 = jnp.maximum(m0, ch)
        t2 = jnp.minimum(m1, t1)
        m1 = jnp.maximum(m1, t1)
        m2 = jnp.maximum(m2, t2)

    c = jnp.max(cacc, axis=0, keepdims=True)

    u = jnp.concatenate([m0, m1, m2], axis=0)
    urow = jax.lax.broadcasted_iota(jnp.int32, u.shape, 0)
    p0 = jnp.max(u, axis=0, keepdims=True)
    a0 = jnp.min(jnp.where(u == p0, urow, _BIG), axis=0, keepdims=True)
    u1 = jnp.where(urow == a0, _NEG_INF, u)
    p1 = jnp.max(u1, axis=0, keepdims=True)
    a1 = jnp.min(jnp.where(u1 == p1, urow, _BIG), axis=0, keepdims=True)
    u2 = jnp.where(urow == a1, _NEG_INF, u1)
    p2 = jnp.max(u2, axis=0, keepdims=True)

    target = jnp.where(c == p0, p1, p0)
    o_ref[...] = ((target - c) / (p0 - p2))[0]


@functools.partial(jax.jit, static_argnames=("block_cols",))
def _dlr_tc(prediction, y, block_cols=2048):
    n, c = prediction.shape
    xt = prediction.T
    return pl.pallas_call(
        _dlr_body,
        grid=(n // block_cols,),
        in_specs=[
            pl.BlockSpec((c, block_cols), lambda i: (0, i)),
            pl.BlockSpec((block_cols,), lambda i: (i,)),
        ],
        out_specs=pl.BlockSpec((block_cols,), lambda i: (i,)),
        out_shape=jax.ShapeDtypeStruct((n,), jnp.float32),
    )(xt, y)


_N = 16384
_C = 1000
_NW = 32
_T = 12288
_NSC = _N - _T
_W = _NSC // _NW
_S = 128
_CB = _W // _S
_RCH = 200
_NRB = _C // _RCH


def _sc_body(xt_hbm, y_hbm, out_hbm,
             b0, b1, b2, b3, b4, y_v, out_v, s0, s1, s2, s3, s4):
    wid = lax.axis_index("s") * 2 + lax.axis_index("c")
    base = _T + wid * _W

    bufs = (b0, b1, b2, b3, b4)
    sems = (s0, s1, s2, s3, s4)

    def copy_for(rb):
        return pltpu.make_async_copy(
            xt_hbm.at[pl.ds(rb * _RCH, _RCH), pl.ds(base, _S)],
            bufs[rb], sems[rb])

    for rb in range(_NRB):
        copy_for(rb).start()
    pltpu.sync_copy(y_hbm.at[pl.ds(base, _W)], y_v)

    neg = jnp.full((16,), _NEG_INF, dtype=jnp.float32)
    carry = [(neg, neg, neg, neg)] * (_S // 16)
    yvs = [y_v[pl.ds(g * 16, 16)] for g in range(_S // 16)]
    for rb in range(_NRB):
        copy_for(rb).wait()
        buf = bufs[rb]
        r0 = rb * _RCH
        _IL = 4
        for g in range(0, _S // 16, _IL):

            def step(r, cN, buf=buf, g=g, r0=r0):
                out = []
                for j in range(_IL):
                    m0, m1, m2, cacc = cN[j]
                    v = buf[r, pl.ds((g + j) * 16, 16)]
                    cacc = jnp.where(yvs[g + j] == r0 + r, v, cacc)
                    t1 = jnp.minimum(m0, v)
                    m0 = jnp.maximum(m0, v)
                    t2 = jnp.minimum(m1, t1)
                    m1 = jnp.maximum(m1, t1)
                    m2 = jnp.maximum(m2, t2)
                    out.append((m0, m1, m2, cacc))
                return tuple(out)

            res = lax.fori_loop(
                0, _RCH, step, tuple(carry[g + j] for j in range(_IL)),
                unroll=2)
            for j in range(_IL):
                carry[g + j] = res[j]
    for g in range(_S // 16):
        m0, m1, m2, cacc = carry[g]
        target = jnp.where(cacc == m0, m1, m0)
        out_v[pl.ds(g * 16, 16)] = (target - cacc) / (m0 - m2)

    pltpu.sync_copy(out_v, out_hbm.at[pl.ds(wid * _W, _W)])


def _dlr_sc_call(xt, y):
    mesh = plsc.VectorSubcoreMesh(core_axis_name="c", subcore_axis_name="s")
    k = functools.partial(
        pl.kernel,
        mesh=mesh,
        out_type=jax.ShapeDtypeStruct((_NSC,), jnp.float32),
        scratch_types=(
            [pltpu.VMEM((_RCH, _S), jnp.float32)] * _NRB
            + [pltpu.VMEM((_W,), jnp.int32), pltpu.VMEM((_W,), jnp.float32)]
            + [pltpu.SemaphoreType.DMA] * _NRB
        ),
    )(_sc_body)
    return k(xt, y)


@jax.jit
def _dlr_hybrid(prediction, y):
    n, c = prediction.shape
    xt = prediction.T
    loss_sc = _dlr_sc_call(xt, y)
    block_cols = 2048
    loss_tc = pl.pallas_call(
        _dlr_body,
        grid=(_T // block_cols,),
        in_specs=[
            pl.BlockSpec((c, block_cols), lambda i: (0, i)),
            pl.BlockSpec((block_cols,), lambda i: (i,)),
        ],
        out_specs=pl.BlockSpec((block_cols,), lambda i: (i,)),
        out_shape=jax.ShapeDtypeStruct((_T,), jnp.float32),
    )(xt, y)
    return jnp.concatenate([loss_tc, loss_sc])


def kernel(prediction, y):
    return _dlr_hybrid(prediction, y)


# speedup vs baseline: 1.5831x; 1.5831x over previous
import functools

import jax
import jax.numpy as jnp
from jax.experimental import pallas as pl

_NEG_INF = float("-inf")
_BIG = 1 << 30
_SUB = 8


def _dlr_body(xt_ref, y_ref, o_ref):
    xt = xt_ref[...]
    yv = y_ref[...][None, :]
    C, B = xt.shape
    n_chunks = C // _SUB

    subl = jax.lax.broadcasted_iota(jnp.int32, (_SUB, B), 0)
    neg = jnp.full((_SUB, B), _NEG_INF, dtype=jnp.float32)

    ch = xt[:_SUB, :]
    m0, m1, m2 = ch, neg, neg
    cacc = jnp.where(subl == yv, ch, _NEG_INF)

    for k in range(1, n_chunks):
        ch = xt[k * _SUB:(k + 1) * _SUB, :]
        cacc = jnp.where(subl == yv - (k * _SUB), ch, cacc)
        t1 = jnp.minimum(m0, ch)
        m0 = jnp.maximum(m0, ch)
        t2 = jnp.minimum(m1, t1)
        m1 = jnp.maximum(m1, t1)
        m2 = jnp.maximum(m2, t2)

    c = jnp.max(cacc, axis=0, keepdims=True)

    u = jnp.concatenate([m0, m1, m2], axis=0)
    urow = jax.lax.broadcasted_iota(jnp.int32, u.shape, 0)
    p0 = jnp.max(u, axis=0, keepdims=True)
    a0 = jnp.min(jnp.where(u == p0, urow, _BIG), axis=0, keepdims=True)
    u1 = jnp.where(urow == a0, _NEG_INF, u)
    p1 = jnp.max(u1, axis=0, keepdims=True)
    a1 = jnp.min(jnp.where(u1 == p1, urow, _BIG), axis=0, keepdims=True)
    u2 = jnp.where(urow == a1, _NEG_INF, u1)
    p2 = jnp.max(u2, axis=0, keepdims=True)

    target = jnp.where(c == p0, p1, p0)
    o_ref[...] = ((target - c) / (p0 - p2))[0]


@functools.partial(jax.jit, static_argnames=("block_cols",))
def _dlr_tc(prediction, y, block_cols=2048):
    n, c = prediction.shape
    xt = prediction.T
    return pl.pallas_call(
        _dlr_body,
        grid=(n // block_cols,),
        in_specs=[
            pl.BlockSpec((c, block_cols), lambda i: (0, i)),
            pl.BlockSpec((block_cols,), lambda i: (i,)),
        ],
        out_specs=pl.BlockSpec((block_cols,), lambda i: (i,)),
        out_shape=jax.ShapeDtypeStruct((n,), jnp.float32),
    )(xt, y)


def kernel(prediction, y):
    return _dlr_tc(prediction, y)
